# baseline (speedup 1.0000x reference)
import functools

import jax
import jax.numpy as jnp
from jax import lax
from jax.experimental import pallas as pl
from jax.experimental.pallas import tpu as pltpu
from jax.experimental.pallas import tpu_sc as plsc

_HID = 64
_B, _S = 1024, 200
_NC, _NS = 2, 16
_NW = _NC * _NS
_CHUNK = 128
_NBBLK = _B // _CHUNK
_SPERW = _S // (_NW // _NBBLK)
_NBUF = 5
_NG = _SPERW // _NBUF
_LANE = 16


def _sc_embed_body(ids_hbm, cw_hbm, table_hbm, out_hbm,
                   idx_v, g0, g1, g2, g3, g4, s0_, s1_, s2_, s3_, s4_,
                   cw_v, gsem, ssem):
    gbufs = [g0, g1, g2, g3, g4]
    sbufs = [s0_, s1_, s2_, s3_, s4_]
    wid = lax.axis_index("s") * _NC + lax.axis_index("c")
    cb = wid % _NBBLK
    s0 = (wid // _NBBLK) * _SPERW

    pltpu.sync_copy(ids_hbm.at[pl.ds(s0, _SPERW), pl.ds(cb, 1)], idx_v)
    pltpu.sync_copy(cw_hbm, cw_v)

    w_vec = cw_v[_HID]
    row_idx = [lax.iota(jnp.int32, 16) + (g * _LANE) for g in range(8)]

    def gather_start(k, b):
        pltpu.make_async_copy(
            table_hbm.at[idx_v.at[k, 0]], gbufs[b], gsem.at[b]).start()

    def gather_wait(b):
        pltpu.make_async_copy(
            table_hbm.at[idx_v.at[0, 0]], gbufs[b], gsem.at[b]).wait()

    def store_start(k, b):
        pltpu.make_async_copy(
            sbufs[b], out_hbm.at[s0 + k, slice(None), cb], ssem.at[b]).start()

    def store_wait(b):
        pltpu.make_async_copy(
            sbufs[b], out_hbm.at[s0, slice(None), cb], ssem.at[b]).wait()

    def fma_transpose(b):
        gb = gbufs[b]
        sb = sbufs[b]

        def body(r_hi, carry):
            for r_lo in range(8):
                h = r_hi * 8 + r_lo
                cv = cw_v[h]
                hv = jnp.full((16,), h, jnp.int32)
                for g in range(8):
                    v = plsc.load_gather(gb, [row_idx[g], hv])
                    sb[r_hi, r_lo, pl.ds(g * _LANE, _LANE)] = v * w_vec + cv
            return carry

        lax.fori_loop(0, 8, body, 0, unroll=False)

    for b in range(_NBUF):
        gather_start(b, b)

    def outer(g, carry):
        for b in range(_NBUF):
            k = g * _NBUF + b
            gather_wait(b)

            @pl.when(g > 0)
            def _wait_prev_store():
                store_wait(b)

            fma_transpose(b)
            store_start(k, b)

            @pl.when(g < _NG - 1)
            def _refill():
                gather_start(k + _NBUF, b)
        return carry

    lax.fori_loop(0, _NG, outer, 0, unroll=False)

    for b in range(_NBUF):
        store_wait(b)


_embed_call = functools.partial(
    pl.kernel,
    out_type=jax.ShapeDtypeStruct((_S, 8, _NBBLK, 8, _CHUNK), jnp.float32),
    mesh=plsc.VectorSubcoreMesh(core_axis_name="c", subcore_axis_name="s"),
    compiler_params=pltpu.CompilerParams(
        use_tc_tiling_on_sc=False, needs_layout_passes=False),
    scratch_types=[
        pltpu.VMEM((_SPERW, 1, _CHUNK), jnp.int32),
        *[pltpu.VMEM((_CHUNK, _HID), jnp.float32) for _ in range(_NBUF)],
        *[pltpu.VMEM((8, 8, _CHUNK), jnp.float32) for _ in range(_NBUF)],
        pltpu.VMEM((_HID + 1, _LANE), jnp.float32),
        pltpu.SemaphoreType.DMA((_NBUF,)),
        pltpu.SemaphoreType.DMA((_NBUF,)),
    ],
)(_sc_embed_body)


def kernel(input_features, word_table, tt_table, singleton_weight, singleton_bias):
    ids = input_features.T.reshape(_S, _NBBLK, _CHUNK).astype(jnp.int32)
    w = singleton_weight[0].astype(jnp.float32)
    c = tt_table[0].astype(jnp.float32) * w + singleton_bias[0].astype(jnp.float32)
    cw = jnp.concatenate(
        [jnp.tile(c[:, None], (1, _LANE)),
         jnp.full((1, _LANE), w, jnp.float32)], axis=0)
    out5 = _embed_call(ids, cw, word_table.astype(jnp.float32))
    return out5.transpose(2, 4, 0, 1, 3).reshape(_B, _S, _HID)


# speedup vs baseline: 1.2846x; 1.2846x over previous
import functools

import jax
import jax.numpy as jnp
from jax import lax
from jax.experimental import pallas as pl
from jax.experimental.pallas import tpu as pltpu
from jax.experimental.pallas import tpu_sc as plsc

_HID = 64
_B, _S = 1024, 200
_NC, _NS = 2, 16
_NW = _NC * _NS
_CHUNK = 128
_NBBLK = _B // _CHUNK
_SPERW = _S // (_NW // _NBBLK)
_NBUF = 5
_NG = _SPERW // _NBUF
_LANE = 16


def _sc_embed_body(ids_hbm, cw_hbm, table_hbm, out_hbm,
                   idx_v, g0, g1, g2, g3, g4, s0_, s1_, s2_, s3_, s4_,
                   cw_v, gsem, ssem):
    gbufs = [g0, g1, g2, g3, g4]
    sbufs = [s0_, s1_, s2_, s3_, s4_]
    wid = lax.axis_index("s") * _NC + lax.axis_index("c")
    cb = wid % _NBBLK
    s0 = (wid // _NBBLK) * _SPERW

    pltpu.sync_copy(ids_hbm.at[pl.ds(s0, _SPERW), pl.ds(cb, 1)], idx_v)
    pltpu.sync_copy(cw_hbm, cw_v)

    w_vec = cw_v[pl.ds(_HID, _LANE)]
    c_vecs = [cw_v[pl.ds(g * _LANE, _LANE)] for g in range(4)]
    hidx = [lax.iota(jnp.int32, 16) + (g * _LANE) for g in range(4)]
    ridx_hi = [h >> 3 for h in hidx]
    ridx_lo = [h & 7 for h in hidx]

    def gather_start(k, b):
        pltpu.make_async_copy(
            table_hbm.at[idx_v.at[k, 0]], gbufs[b], gsem.at[b]).start()

    def gather_wait(b):
        pltpu.make_async_copy(
            table_hbm.at[idx_v.at[0, 0]], gbufs[b], gsem.at[b]).wait()

    def store_start(k, b):
        pltpu.make_async_copy(
            sbufs[b].at[:, :, pl.ds(0, _CHUNK)],
            out_hbm.at[s0 + k, slice(None), cb], ssem.at[b]).start()

    def store_wait(b):
        pltpu.make_async_copy(
            sbufs[b].at[:, :, pl.ds(0, _CHUNK)],
            out_hbm.at[s0, slice(None), cb], ssem.at[b]).wait()

    def fma_transpose(b):
        gb = gbufs[b]
        sb = sbufs[b]

        def body(t4, carry):
            for dt in range(4):
                t = t4 * 4 + dt
                lv = jnp.full((16,), t, jnp.int32)
                for g in range(4):
                    v = gb[t, pl.ds(g * _LANE, _LANE)] * w_vec + c_vecs[g]
                    plsc.store_scatter(sb, [ridx_hi[g], ridx_lo[g], lv], v)
            return carry

        lax.fori_loop(0, _CHUNK // 4, body, 0, unroll=False)

    for b in range(_NBUF):
        gather_start(b, b)

    def outer(g, carry):
        for b in range(_NBUF):
            k = g * _NBUF + b
            gather_wait(b)

            @pl.when(g > 0)
            def _wait_prev_store():
                store_wait(b)

            fma_transpose(b)
            store_start(k, b)

            @pl.when(g < _NG - 1)
            def _refill():
                gather_start(k + _NBUF, b)
        return carry

    lax.fori_loop(0, _NG, outer, 0, unroll=False)

    for b in range(_NBUF):
        store_wait(b)


_embed_call = functools.partial(
    pl.kernel,
    out_type=jax.ShapeDtypeStruct((_S, 8, _NBBLK, 8, _CHUNK), jnp.float32),
    mesh=plsc.VectorSubcoreMesh(core_axis_name="c", subcore_axis_name="s"),
    compiler_params=pltpu.CompilerParams(
        use_tc_tiling_on_sc=False, needs_layout_passes=False),
    scratch_types=[
        pltpu.VMEM((_SPERW, 1, _CHUNK), jnp.int32),
        *[pltpu.VMEM((_CHUNK, _HID), jnp.float32) for _ in range(_NBUF)],
        *[pltpu.VMEM((8, 8, _CHUNK + 5), jnp.float32) for _ in range(_NBUF)],
        pltpu.VMEM((_HID + _LANE,), jnp.float32),
        pltpu.SemaphoreType.DMA((_NBUF,)),
        pltpu.SemaphoreType.DMA((_NBUF,)),
    ],
)(_sc_embed_body)


def kernel(input_features, word_table, tt_table, singleton_weight, singleton_bias):
    ids = input_features.T.reshape(_S, _NBBLK, _CHUNK).astype(jnp.int32)
    w = singleton_weight[0].astype(jnp.float32)
    c = tt_table[0].astype(jnp.float32) * w + singleton_bias[0].astype(jnp.float32)
    cw = jnp.concatenate([c, jnp.full((_LANE,), w, jnp.float32)])
    out5 = _embed_call(ids, cw, word_table.astype(jnp.float32))
    return out5.transpose(2, 4, 0, 1, 3).reshape(_B, _S, _HID)


# speedup vs baseline: 1.4309x; 1.1139x over previous
import functools

import jax
import jax.numpy as jnp
from jax import lax
from jax.experimental import pallas as pl
from jax.experimental.pallas import tpu as pltpu
from jax.experimental.pallas import tpu_sc as plsc

_HID = 64
_B, _S = 1024, 200
_NC, _NS = 2, 16
_NW = _NC * _NS
_CHUNK = 128
_NBBLK = _B // _CHUNK
_SPERW = _S // (_NW // _NBBLK)
_NBUF = 5
_NG = _SPERW // _NBUF
_LANE = 16


def _sc_embed_body(ids_hbm, cw_hbm, table_hbm, out_hbm,
                   idx_v, g0, g1, g2, g3, g4, s0_, s1_, s2_, s3_, s4_,
                   cw_v, gsem, ssem):
    gbufs = [g0, g1, g2, g3, g4]
    sbufs = [s0_, s1_, s2_, s3_, s4_]
    wid = lax.axis_index("s") * _NC + lax.axis_index("c")
    cb = wid % _NBBLK
    s0 = (wid // _NBBLK) * _SPERW

    pltpu.sync_copy(ids_hbm.at[pl.ds(s0, _SPERW), pl.ds(cb, 1)], idx_v)
    pltpu.sync_copy(cw_hbm, cw_v)

    w_vec = cw_v[pl.ds(_HID, _LANE)]
    c_vecs = [cw_v[pl.ds(g * _LANE, _LANE)] for g in range(4)]
    hidx = [lax.iota(jnp.int32, 16) + (g * _LANE) for g in range(4)]
    ridx_hi = [h >> 3 for h in hidx]
    ridx_lo = [h & 7 for h in hidx]

    def gather_start(k, b):
        pltpu.make_async_copy(
            table_hbm.at[idx_v.at[k, 0]], gbufs[b], gsem.at[b]).start()

    def gather_wait(b):
        pltpu.make_async_copy(
            table_hbm.at[idx_v.at[0, 0]], gbufs[b], gsem.at[b]).wait()

    def store_start(k, b):
        pltpu.make_async_copy(
            sbufs[b].at[:, :, pl.ds(0, _CHUNK)],
            out_hbm.at[s0 + k, slice(None), cb], ssem.at[b]).start()

    def store_wait(b):
        pltpu.make_async_copy(
            sbufs[b].at[:, :, pl.ds(0, _CHUNK)],
            out_hbm.at[s0, slice(None), cb], ssem.at[b]).wait()

    def fma_transpose(b):
        gb = gbufs[b]
        sb = sbufs[b]

        @plsc.parallel_loop(0, _CHUNK // 4, unroll=2)
        def _body(t4):
            for dt in range(4):
                t = t4 * 4 + dt
                lv = jnp.full((16,), t, jnp.int32)
                for g in range(4):
                    v = gb[t, pl.ds(g * _LANE, _LANE)] * w_vec + c_vecs[g]
                    plsc.store_scatter(sb, [ridx_hi[g], ridx_lo[g], lv], v)

    for b in range(_NBUF):
        gather_start(b, b)

    def outer(g, carry):
        for b in range(_NBUF):
            k = g * _NBUF + b
            gather_wait(b)

            @pl.when(g > 0)
            def _wait_prev_store():
                store_wait(b)

            fma_transpose(b)
            store_start(k, b)

            @pl.when(g < _NG - 1)
            def _refill():
                gather_start(k + _NBUF, b)
        return carry

    lax.fori_loop(0, _NG, outer, 0, unroll=False)

    for b in range(_NBUF):
        store_wait(b)


_embed_call = functools.partial(
    pl.kernel,
    out_type=jax.ShapeDtypeStruct((_S, 8, _NBBLK, 8, _CHUNK), jnp.float32),
    mesh=plsc.VectorSubcoreMesh(core_axis_name="c", subcore_axis_name="s"),
    compiler_params=pltpu.CompilerParams(
        use_tc_tiling_on_sc=False, needs_layout_passes=False),
    scratch_types=[
        pltpu.VMEM((_SPERW, 1, _CHUNK), jnp.int32),
        *[pltpu.VMEM((_CHUNK, _HID), jnp.float32) for _ in range(_NBUF)],
        *[pltpu.VMEM((8, 8, _CHUNK + 5), jnp.float32) for _ in range(_NBUF)],
        pltpu.VMEM((_HID + _LANE,), jnp.float32),
        pltpu.SemaphoreType.DMA((_NBUF,)),
        pltpu.SemaphoreType.DMA((_NBUF,)),
    ],
)(_sc_embed_body)


def kernel(input_features, word_table, tt_table, singleton_weight, singleton_bias):
    ids = input_features.T.reshape(_S, _NBBLK, _CHUNK).astype(jnp.int32)
    w = singleton_weight[0].astype(jnp.float32)
    c = tt_table[0].astype(jnp.float32) * w + singleton_bias[0].astype(jnp.float32)
    cw = jnp.concatenate([c, jnp.full((_LANE,), w, jnp.float32)])
    out5 = _embed_call(ids, cw, word_table.astype(jnp.float32))
    return out5.transpose(2, 4, 0, 1, 3).reshape(_B, _S, _HID)
